# Initial kernel scaffold; baseline (speedup 1.0000x reference)
import jax
import jax.numpy as jnp
from jax.experimental import pallas as pl


def kernel(rel_embed, rel_embed_in, rel_embed_out, w_in, w_out, gamma, beta, edge_index, edge_type):
    raise NotImplementedError("write your pallas kernel here")


# speedup vs baseline: 2.3710x; 2.3710x over previous
import functools

import jax
import jax.numpy as jnp
from jax.experimental import pallas as pl
from jax.experimental.pallas import tpu as pltpu

N_ENT = 10000
N_REL = 500
D = 128
ROW_BLK = 2000


def _mm_bn_kernel(ri_ref, ro_ref, wi_ref, wo_ref, sin_ref, sout_ref,
                  degi_ref, dego_ref, x_ref, stats_ref, ti_ref, to_ref, acc_ref):
    step = pl.program_id(0)

    @pl.when(step == 0)
    def _():
        ti_ref[...] = jnp.dot(ri_ref[...], wi_ref[...],
                              preferred_element_type=jnp.float32)
        to_ref[...] = jnp.dot(ro_ref[...], wo_ref[...],
                              preferred_element_type=jnp.float32)
        acc_ref[...] = jnp.zeros_like(acc_ref)

    degi = degi_ref[...]
    dego = dego_ref[...]
    dinv_i = jnp.where(degi > 0, jax.lax.rsqrt(degi), 0.0)
    dinv_o = jnp.where(dego > 0, jax.lax.rsqrt(dego), 0.0)
    xi = jnp.dot(sin_ref[...], ti_ref[...], preferred_element_type=jnp.float32)
    xo = jnp.dot(sout_ref[...], to_ref[...], preferred_element_type=jnp.float32)
    x = 0.5 * (dinv_i * xi + dinv_o * xo)
    x_ref[...] = x
    acc_ref[0, :] += jnp.sum(x, axis=0)
    acc_ref[1, :] += jnp.sum(x * x, axis=0)

    @pl.when(step == pl.num_programs(0) - 1)
    def _():
        stats_ref[...] = acc_ref[...]


def _bn_apply_kernel(x_ref, stats_ref, gamma_ref, beta_ref, out_ref):
    mean = stats_ref[0, :] * (1.0 / N_ENT)
    var = stats_ref[1, :] * (1.0 / N_ENT) - mean * mean
    scale = gamma_ref[...] * jax.lax.rsqrt(var + 1e-5)
    out_ref[...] = jnp.tanh((x_ref[...] - mean) * scale + beta_ref[...])


def _dense_stage(rel_embed_in, rel_embed_out, w_in, w_out, gamma, beta,
                 s_in, s_out, deg_in, deg_out):
    nb = N_ENT // ROW_BLK
    x, stats = pl.pallas_call(
        _mm_bn_kernel,
        grid=(nb,),
        in_specs=[
            pl.BlockSpec((N_REL, D), lambda i: (0, 0)),
            pl.BlockSpec((N_REL, D), lambda i: (0, 0)),
            pl.BlockSpec((D, D), lambda i: (0, 0)),
            pl.BlockSpec((D, D), lambda i: (0, 0)),
            pl.BlockSpec((ROW_BLK, N_REL), lambda i: (i, 0)),
            pl.BlockSpec((ROW_BLK, N_REL), lambda i: (i, 0)),
            pl.BlockSpec((ROW_BLK, 1), lambda i: (i, 0)),
            pl.BlockSpec((ROW_BLK, 1), lambda i: (i, 0)),
        ],
        out_specs=[
            pl.BlockSpec((ROW_BLK, D), lambda i: (i, 0)),
            pl.BlockSpec((2, D), lambda i: (0, 0)),
        ],
        out_shape=[
            jax.ShapeDtypeStruct((N_ENT, D), jnp.float32),
            jax.ShapeDtypeStruct((2, D), jnp.float32),
        ],
        scratch_shapes=[
            pltpu.VMEM((N_REL, D), jnp.float32),
            pltpu.VMEM((N_REL, D), jnp.float32),
            pltpu.VMEM((2, D), jnp.float32),
        ],
    )(rel_embed_in, rel_embed_out, w_in, w_out, s_in, s_out,
      deg_in[:, None], deg_out[:, None])

    res = pl.pallas_call(
        _bn_apply_kernel,
        grid=(nb,),
        in_specs=[
            pl.BlockSpec((ROW_BLK, D), lambda i: (i, 0)),
            pl.BlockSpec((2, D), lambda i: (0, 0)),
            pl.BlockSpec((D,), lambda i: (0,)),
            pl.BlockSpec((D,), lambda i: (0,)),
        ],
        out_specs=pl.BlockSpec((ROW_BLK, D), lambda i: (i, 0)),
        out_shape=jax.ShapeDtypeStruct((N_ENT, D), jnp.float32),
    )(x, stats, gamma, beta)
    return res


def kernel(rel_embed, rel_embed_in, rel_embed_out, w_in, w_out, gamma, beta,
           edge_index, edge_type):
    e = edge_index.shape[1]
    es = e // 2
    row = edge_index[0]
    col = edge_index[1]
    one = jnp.ones((es,), jnp.float32)

    deg_in = jnp.zeros((N_ENT,), jnp.float32).at[row[:es]].add(one)
    deg_out = jnp.zeros((N_ENT,), jnp.float32).at[row[es:]].add(one)
    dinv_in = jnp.where(deg_in > 0, jax.lax.rsqrt(deg_in), 0.0)
    dinv_out = jnp.where(deg_out > 0, jax.lax.rsqrt(deg_out), 0.0)
    z_in = dinv_in[col[:es]]
    z_out = dinv_out[col[es:]]
    s_in = jnp.zeros((N_ENT, N_REL), jnp.float32).at[row[:es], edge_type[:es]].add(z_in)
    s_out = jnp.zeros((N_ENT, N_REL), jnp.float32).at[row[es:], edge_type[es:]].add(z_out)

    res = _dense_stage(rel_embed_in, rel_embed_out, w_in, w_out, gamma, beta,
                       s_in, s_out, deg_in, deg_out)
    return (res, rel_embed)


# speedup vs baseline: 23.2468x; 9.8048x over previous
import functools

import jax
import jax.numpy as jnp
from jax import lax
from jax.experimental import pallas as pl
from jax.experimental.pallas import tpu as pltpu
from jax.experimental.pallas import tpu_sc as plsc

N_ENT = 10000
N_REL = 500
RELP = 512
D = 128
ROW_BLK = 2000

NT = 16
E_HALF = 160000
EPT = E_HALF // NT
NBATCH = 79
EPT_PAD = NBATCH * 128

NCHUNK = 4
CH_N = N_ENT // NCHUNK
CH_FLAT = CH_N * RELP
CH_TOT = CH_FLAT + NT * 128
CH_ZERO_PT = CH_TOT // NT
CH_RD_PT = CH_FLAT // NT
RD_UNIT = 5000
ZU = 5008

DEG_TOT = 12288
DEG_PT = 10240
BIGKEY = 1 << 30


def _sc_scatter_kernel(ei_hbm, et_hbm, s_hbm, deg_hbm,
                       keyb, auxb, zb, zerob, idxb, onesb, tmpb, rdb,
                       s_chunk, deg_hist):
    c = lax.axis_index("c")
    t = lax.axis_index("s")
    ebase = c * E_HALF + t * EPT
    lanes = lax.iota(jnp.int32, 16)

    def zb_body(i, _):
        zerob[pl.ds(i * 16, 16)] = jnp.zeros((16,), jnp.float32)
        return 0
    lax.fori_loop(0, ZU // 16, zb_body, 0)
    for g in range(8):
        onesb[pl.ds(g * 16, 16)] = jnp.ones((16,), jnp.float32)

    pltpu.sync_copy(ei_hbm.at[pl.ds(ebase, EPT)], keyb.at[pl.ds(0, EPT)])
    for p in range(7):
        keyb[pl.ds(EPT + p * 16, 16)] = N_ENT + t * 128 + p * 16 + lanes

    pltpu.sync_copy(zerob.at[pl.ds(0, 768)],
                    deg_hist.at[pl.ds(t * 768, 768)])
    plsc.subcore_barrier()

    def hist_body(b, _):
        for g in range(8):
            idxb[pl.ds(g * 16, 16)] = keyb[pl.ds(b * 128 + g * 16, 16)]
        pltpu.sync_copy(onesb, deg_hist.at[idxb], add=True)
        return 0
    lax.fori_loop(0, NBATCH, hist_body, 0)
    plsc.subcore_barrier()

    pltpu.sync_copy(deg_hist.at[pl.ds(t * 640, 640)], tmpb)
    pltpu.sync_copy(tmpb, deg_hbm.at[pl.ds(c * DEG_PT + t * 640, 640)])

    def dinv_body(i, _):
        x = tmpb[pl.ds(i * 16, 16)]
        y = jnp.full((16,), 1.0, jnp.float32)
        for k in range(1, 11):
            y = jnp.where(x >= float(0.5 * 4 ** k), float(2.0 ** (-k)), y)
        for _ in range(5):
            y = y * (1.5 - 0.5 * x * y * y)
        tmpb[pl.ds(i * 16, 16)] = jnp.where(x >= 1.0, y, 0.0)
        return 0
    lax.fori_loop(0, 640 // 16, dinv_body, 0)
    pltpu.sync_copy(tmpb, deg_hist.at[pl.ds(t * 640, 640)])
    plsc.subcore_barrier()

    for p in range(7):
        auxb[pl.ds(EPT + p * 16, 16)] = jnp.zeros((16,), jnp.int32)
    pltpu.sync_copy(ei_hbm.at[pl.ds(2 * E_HALF + ebase, EPT)],
                    auxb.at[pl.ds(0, EPT)])

    def z_body(b, _):
        for g in range(8):
            idxb[pl.ds(g * 16, 16)] = auxb[pl.ds(b * 128 + g * 16, 16)]
        pltpu.sync_copy(deg_hist.at[idxb], zb.at[pl.ds(b * 128, 128)])
        return 0
    lax.fori_loop(0, NBATCH, z_body, 0)

    pltpu.sync_copy(et_hbm.at[pl.ds(ebase, EPT)], auxb.at[pl.ds(0, EPT)])

    def key_body(i, _):
        keyb[pl.ds(i * 16, 16)] = (keyb[pl.ds(i * 16, 16)] * RELP
                                   + auxb[pl.ds(i * 16, 16)])
        return 0
    lax.fori_loop(0, EPT // 16, key_body, 0)
    for p in range(7):
        keyb[pl.ds(EPT + p * 16, 16)] = jnp.full((16,), BIGKEY, jnp.int32)

    def chunk_body(ci, _):
        cbase = ci * CH_FLAT

        def zero_body(j, _):
            pltpu.sync_copy(zerob.at[pl.ds(0, ZU)],
                            s_chunk.at[pl.ds(t * CH_ZERO_PT + j * ZU, ZU)])
            return 0
        lax.fori_loop(0, CH_ZERO_PT // ZU, zero_body, 0)
        plsc.subcore_barrier()

        def batch_body(b, _):
            for g in range(8):
                k16 = keyb[pl.ds(b * 128 + g * 16, 16)]
                local = k16 - cbase
                m = (local >= 0) & (local < CH_FLAT)
                dmp = CH_FLAT + t * 128 + g * 16 + lanes
                idxb[pl.ds(g * 16, 16)] = jnp.where(m, local, dmp)
            pltpu.sync_copy(zb.at[pl.ds(b * 128, 128)],
                            s_chunk.at[idxb], add=True)
            return 0
        lax.fori_loop(0, NBATCH, batch_body, 0)
        plsc.subcore_barrier()

        def rd_body(j, _):
            off = t * CH_RD_PT + j * RD_UNIT
            pltpu.sync_copy(s_chunk.at[pl.ds(off, RD_UNIT)],
                            rdb.at[pl.ds(0, RD_UNIT)])
            pltpu.sync_copy(rdb.at[pl.ds(0, RD_UNIT)],
                            s_hbm.at[pl.ds(c * NCHUNK * CH_FLAT
                                           + ci * CH_FLAT + off, RD_UNIT)])
            return 0
        lax.fori_loop(0, CH_RD_PT // RD_UNIT, rd_body, 0)
        plsc.subcore_barrier()
        return 0
    lax.fori_loop(0, NCHUNK, chunk_body, 0)


def _sc_scatter(edge_index, edge_type):
    mesh = plsc.VectorSubcoreMesh(core_axis_name="c", subcore_axis_name="s")
    kern = functools.partial(
        pl.kernel,
        mesh=mesh,
        out_type=[
            jax.ShapeDtypeStruct((2 * NCHUNK * CH_FLAT,), jnp.float32),
            jax.ShapeDtypeStruct((2 * DEG_PT,), jnp.float32),
        ],
        scratch_types=[
            pltpu.VMEM((EPT_PAD,), jnp.int32),
            pltpu.VMEM((EPT_PAD,), jnp.int32),
            pltpu.VMEM((EPT_PAD,), jnp.float32),
            pltpu.VMEM((ZU,), jnp.float32),
            pltpu.VMEM((128,), jnp.int32),
            pltpu.VMEM((128,), jnp.float32),
            pltpu.VMEM((640,), jnp.float32),
            pltpu.VMEM((RD_UNIT,), jnp.float32),
            pltpu.VMEM_SHARED((CH_TOT,), jnp.float32),
            pltpu.VMEM_SHARED((DEG_TOT,), jnp.float32),
        ],
    )(_sc_scatter_kernel)
    return kern(edge_index.reshape(-1), edge_type)


def _mm_bn_kernel(ri_ref, ro_ref, wi_ref, wo_ref, sin_ref, sout_ref,
                  degi_ref, dego_ref, x_ref, stats_ref, ti_ref, to_ref, acc_ref):
    step = pl.program_id(0)

    @pl.when(step == 0)
    def _():
        ti_ref[...] = jnp.dot(ri_ref[...], wi_ref[...],
                              preferred_element_type=jnp.float32)
        to_ref[...] = jnp.dot(ro_ref[...], wo_ref[...],
                              preferred_element_type=jnp.float32)
        acc_ref[...] = jnp.zeros_like(acc_ref)

    degi = degi_ref[...]
    dego = dego_ref[...]
    dinv_i = jnp.where(degi > 0, jax.lax.rsqrt(degi), 0.0)
    dinv_o = jnp.where(dego > 0, jax.lax.rsqrt(dego), 0.0)
    xi = jnp.dot(sin_ref[...], ti_ref[...], preferred_element_type=jnp.float32)
    xo = jnp.dot(sout_ref[...], to_ref[...], preferred_element_type=jnp.float32)
    x = 0.5 * (dinv_i * xi + dinv_o * xo)
    x_ref[...] = x
    acc_ref[0, :] += jnp.sum(x, axis=0)
    acc_ref[1, :] += jnp.sum(x * x, axis=0)

    @pl.when(step == pl.num_programs(0) - 1)
    def _():
        stats_ref[...] = acc_ref[...]


def _bn_apply_kernel(x_ref, stats_ref, gamma_ref, beta_ref, out_ref):
    mean = stats_ref[0, :] * (1.0 / N_ENT)
    var = stats_ref[1, :] * (1.0 / N_ENT) - mean * mean
    scale = gamma_ref[...] * jax.lax.rsqrt(var + 1e-5)
    out_ref[...] = jnp.tanh((x_ref[...] - mean) * scale + beta_ref[...])


def _dense_stage(rel_in_p, rel_out_p, w_in, w_out, gamma, beta,
                 s_in, s_out, deg_in, deg_out):
    nb = N_ENT // ROW_BLK
    x, stats = pl.pallas_call(
        _mm_bn_kernel,
        grid=(nb,),
        in_specs=[
            pl.BlockSpec((RELP, D), lambda i: (0, 0)),
            pl.BlockSpec((RELP, D), lambda i: (0, 0)),
            pl.BlockSpec((D, D), lambda i: (0, 0)),
            pl.BlockSpec((D, D), lambda i: (0, 0)),
            pl.BlockSpec((ROW_BLK, RELP), lambda i: (i, 0)),
            pl.BlockSpec((ROW_BLK, RELP), lambda i: (i, 0)),
            pl.BlockSpec((ROW_BLK, 1), lambda i: (i, 0)),
            pl.BlockSpec((ROW_BLK, 1), lambda i: (i, 0)),
        ],
        out_specs=[
            pl.BlockSpec((ROW_BLK, D), lambda i: (i, 0)),
            pl.BlockSpec((2, D), lambda i: (0, 0)),
        ],
        out_shape=[
            jax.ShapeDtypeStruct((N_ENT, D), jnp.float32),
            jax.ShapeDtypeStruct((2, D), jnp.float32),
        ],
        scratch_shapes=[
            pltpu.VMEM((RELP, D), jnp.float32),
            pltpu.VMEM((RELP, D), jnp.float32),
            pltpu.VMEM((2, D), jnp.float32),
        ],
    )(rel_in_p, rel_out_p, w_in, w_out, s_in, s_out,
      deg_in[:, None], deg_out[:, None])

    res = pl.pallas_call(
        _bn_apply_kernel,
        grid=(nb,),
        in_specs=[
            pl.BlockSpec((ROW_BLK, D), lambda i: (i, 0)),
            pl.BlockSpec((2, D), lambda i: (0, 0)),
            pl.BlockSpec((D,), lambda i: (0,)),
            pl.BlockSpec((D,), lambda i: (0,)),
        ],
        out_specs=pl.BlockSpec((ROW_BLK, D), lambda i: (i, 0)),
        out_shape=jax.ShapeDtypeStruct((N_ENT, D), jnp.float32),
    )(x, stats, gamma, beta)
    return res


def kernel(rel_embed, rel_embed_in, rel_embed_out, w_in, w_out, gamma, beta,
           edge_index, edge_type):
    s_flat, deg = _sc_scatter(edge_index, edge_type)
    s_flat = s_flat.reshape(2, N_ENT, RELP)
    s_in = s_flat[0]
    s_out = s_flat[1]
    deg = deg.reshape(2, DEG_PT)
    deg_in = deg[0, :N_ENT]
    deg_out = deg[1, :N_ENT]
    pad = ((0, RELP - N_REL), (0, 0))
    rel_in_p = jnp.pad(rel_embed_in, pad)
    rel_out_p = jnp.pad(rel_embed_out, pad)
    res = _dense_stage(rel_in_p, rel_out_p, w_in, w_out, gamma, beta,
                       s_in, s_out, deg_in, deg_out)
    return (res, rel_embed)


# speedup vs baseline: 34.9287x; 1.5025x over previous
import functools

import jax
import jax.numpy as jnp
from jax import lax
from jax.experimental import pallas as pl
from jax.experimental.pallas import tpu as pltpu
from jax.experimental.pallas import tpu_sc as plsc

N_ENT = 10000
N_REL = 500
D = 128
ROW_BLK = 2000
NPAD = 10240

NT = 16
E_HALF = 160000
EPT = E_HALF // NT
NBATCH = 79
EPT_PAD = NBATCH * 128

NCHUNK = 4
CH_N = NPAD // NCHUNK
CBSZ = CH_N * D
CH_FLAT = 4 * CBSZ
CH_TOT = CH_FLAT + NT * 128
CH_ZERO_PT = CH_TOT // NT
ZU = CH_ZERO_PT // 16
CB_PT = CBSZ // NT
RD_UNIT = CB_PT // 2
SHALF = NPAD * D

DEG_TOT = 12288
DEG_PT = 10240
BIGKEY = 1 << 30


def _sc_scatter_kernel(ei_hbm, et_hbm,
                       s00, s01, s02, s03, s10, s11, s12, s13, deg_hbm,
                       keyb, auxb, zb, zerob, idxb, onesb, tmpb, rdb,
                       s_chunk, deg_hist):
    c = lax.axis_index("c")
    t = lax.axis_index("s")
    ebase = c * E_HALF + t * EPT
    lanes = lax.iota(jnp.int32, 16)

    def zb_body(i, _):
        zerob[pl.ds(i * 16, 16)] = jnp.zeros((16,), jnp.float32)
        return 0
    lax.fori_loop(0, ZU // 16, zb_body, 0)
    for g in range(8):
        onesb[pl.ds(g * 16, 16)] = jnp.ones((16,), jnp.float32)

    pltpu.sync_copy(ei_hbm.at[pl.ds(ebase, EPT)], keyb.at[pl.ds(0, EPT)])
    for p in range(7):
        keyb[pl.ds(EPT + p * 16, 16)] = N_ENT + t * 128 + p * 16 + lanes

    pltpu.sync_copy(zerob.at[pl.ds(0, 768)],
                    deg_hist.at[pl.ds(t * 768, 768)])
    plsc.subcore_barrier()

    def hist_body(b, _):
        for g in range(8):
            idxb[pl.ds(g * 16, 16)] = keyb[pl.ds(b * 128 + g * 16, 16)]
        pltpu.sync_copy(onesb, deg_hist.at[idxb], add=True)
        return 0
    lax.fori_loop(0, NBATCH, hist_body, 0)
    plsc.subcore_barrier()

    pltpu.sync_copy(deg_hist.at[pl.ds(t * 640, 640)], tmpb)
    pltpu.sync_copy(tmpb, deg_hbm.at[pl.ds(c * DEG_PT + t * 640, 640)])

    def dinv_body(i, _):
        x = tmpb[pl.ds(i * 16, 16)]
        y = jnp.full((16,), 1.0, jnp.float32)
        for k in range(1, 11):
            y = jnp.where(x >= float(0.5 * 4 ** k), float(2.0 ** (-k)), y)
        for _ in range(5):
            y = y * (1.5 - 0.5 * x * y * y)
        tmpb[pl.ds(i * 16, 16)] = jnp.where(x >= 1.0, y, 0.0)
        return 0
    lax.fori_loop(0, 640 // 16, dinv_body, 0)
    pltpu.sync_copy(tmpb, deg_hist.at[pl.ds(t * 640, 640)])
    plsc.subcore_barrier()

    for p in range(7):
        auxb[pl.ds(EPT + p * 16, 16)] = jnp.zeros((16,), jnp.int32)
    pltpu.sync_copy(ei_hbm.at[pl.ds(2 * E_HALF + ebase, EPT)],
                    auxb.at[pl.ds(0, EPT)])

    def z_body(b, _):
        for g in range(8):
            idxb[pl.ds(g * 16, 16)] = auxb[pl.ds(b * 128 + g * 16, 16)]
        pltpu.sync_copy(deg_hist.at[idxb], zb.at[pl.ds(b * 128, 128)])
        return 0
    lax.fori_loop(0, NBATCH, z_body, 0)

    pltpu.sync_copy(et_hbm.at[pl.ds(ebase, EPT)], auxb.at[pl.ds(0, EPT)])

    def key_body(i, _):
        ty = auxb[pl.ds(i * 16, 16)]
        keyb[pl.ds(i * 16, 16)] = (
            lax.shift_left(keyb[pl.ds(i * 16, 16)], 7) + (ty & 127))
        auxb[pl.ds(i * 16, 16)] = lax.shift_right_logical(ty, 7)
        return 0
    lax.fori_loop(0, EPT // 16, key_body, 0)
    for p in range(7):
        keyb[pl.ds(EPT + p * 16, 16)] = jnp.full((16,), BIGKEY, jnp.int32)
        auxb[pl.ds(EPT + p * 16, 16)] = jnp.zeros((16,), jnp.int32)

    def chunk_body(ci, _):
        cbase = ci * CBSZ

        def zero_body(j, _):
            pltpu.sync_copy(zerob.at[pl.ds(0, ZU)],
                            s_chunk.at[pl.ds(t * CH_ZERO_PT + j * ZU, ZU)])
            return 0
        lax.fori_loop(0, CH_ZERO_PT // ZU, zero_body, 0)
        plsc.subcore_barrier()

        def batch_body(b, _):
            for g in range(8):
                k16 = keyb[pl.ds(b * 128 + g * 16, 16)]
                cb16 = auxb[pl.ds(b * 128 + g * 16, 16)]
                local = k16 - cbase
                m = (local >= 0) & (local < CBSZ)
                dmp = CH_FLAT + t * 128 + g * 16 + lanes
                idxb[pl.ds(g * 16, 16)] = jnp.where(m, local + cb16 * CBSZ,
                                                    dmp)
            pltpu.sync_copy(zb.at[pl.ds(b * 128, 128)],
                            s_chunk.at[idxb], add=True)
            return 0
        lax.fori_loop(0, NBATCH, batch_body, 0)
        plsc.subcore_barrier()

        for cb, (sa, sb_) in enumerate(((s00, s10), (s01, s11),
                                        (s02, s12), (s03, s13))):
            def rd_body(j, _, cb=cb, sa=sa, sb_=sb_):
                src = cb * CBSZ + t * CB_PT + j * RD_UNIT
                dst = ci * CBSZ + t * CB_PT + j * RD_UNIT
                pltpu.sync_copy(s_chunk.at[pl.ds(src, RD_UNIT)],
                                rdb.at[pl.ds(0, RD_UNIT)])

                @pl.when(c == 0)
                def _():
                    pltpu.sync_copy(rdb.at[pl.ds(0, RD_UNIT)],
                                    sa.at[pl.ds(dst, RD_UNIT)])

                @pl.when(c == 1)
                def _():
                    pltpu.sync_copy(rdb.at[pl.ds(0, RD_UNIT)],
                                    sb_.at[pl.ds(dst, RD_UNIT)])
                return 0
            lax.fori_loop(0, CB_PT // RD_UNIT, rd_body, 0)
        plsc.subcore_barrier()
        return 0
    lax.fori_loop(0, NCHUNK, chunk_body, 0)


def _sc_scatter(edge_index, edge_type):
    mesh = plsc.VectorSubcoreMesh(core_axis_name="c", subcore_axis_name="s")
    kern = functools.partial(
        pl.kernel,
        mesh=mesh,
        out_type=(
            [jax.ShapeDtypeStruct((SHALF,), jnp.float32) for _ in range(8)]
            + [jax.ShapeDtypeStruct((2 * DEG_PT,), jnp.float32)]),
        scratch_types=[
            pltpu.VMEM((EPT_PAD,), jnp.int32),
            pltpu.VMEM((EPT_PAD,), jnp.int32),
            pltpu.VMEM((EPT_PAD,), jnp.float32),
            pltpu.VMEM((ZU,), jnp.float32),
            pltpu.VMEM((128,), jnp.int32),
            pltpu.VMEM((128,), jnp.float32),
            pltpu.VMEM((640,), jnp.float32),
            pltpu.VMEM((RD_UNIT,), jnp.float32),
            pltpu.VMEM_SHARED((CH_TOT,), jnp.float32),
            pltpu.VMEM_SHARED((DEG_TOT,), jnp.float32),
        ],
    )(_sc_scatter_kernel)
    return kern(edge_index.reshape(-1), edge_type)


def _mm_bn_kernel(ri_ref, ro_ref, wi_ref, wo_ref,
                  si0, si1, si2, si3, so0, so1, so2, so3,
                  degi_ref, dego_ref, x_ref, stats_ref, ti_ref, to_ref,
                  acc_ref):
    step = pl.program_id(0)

    @pl.when(step == 0)
    def _():
        ti_ref[...] = jnp.dot(ri_ref[...], wi_ref[...],
                              preferred_element_type=jnp.float32)
        to_ref[...] = jnp.dot(ro_ref[...], wo_ref[...],
                              preferred_element_type=jnp.float32)
        acc_ref[...] = jnp.zeros_like(acc_ref)

    degi = degi_ref[...]
    dego = dego_ref[...]
    dinv_i = jnp.where(degi > 0, jax.lax.rsqrt(degi), 0.0)
    dinv_o = jnp.where(dego > 0, jax.lax.rsqrt(dego), 0.0)
    xi = jnp.dot(si0[...], ti_ref[pl.ds(0, 128), :],
                 preferred_element_type=jnp.float32)
    xo = jnp.dot(so0[...], to_ref[pl.ds(0, 128), :],
                 preferred_element_type=jnp.float32)
    for cb, (si, so) in enumerate(((si1, so1), (si2, so2), (si3, so3)),
                                  start=1):
        xi = xi + jnp.dot(si[...], ti_ref[pl.ds(cb * 128, 128), :],
                          preferred_element_type=jnp.float32)
        xo = xo + jnp.dot(so[...], to_ref[pl.ds(cb * 128, 128), :],
                          preferred_element_type=jnp.float32)
    x = 0.5 * (dinv_i * xi + dinv_o * xo)
    x_ref[...] = x
    acc_ref[0, :] += jnp.sum(x, axis=0)
    acc_ref[1, :] += jnp.sum(x * x, axis=0)

    @pl.when(step == pl.num_programs(0) - 1)
    def _():
        stats_ref[...] = acc_ref[...]


def _bn_apply_kernel(x_ref, stats_ref, gamma_ref, beta_ref, out_ref):
    mean = stats_ref[0, :] * (1.0 / N_ENT)
    var = stats_ref[1, :] * (1.0 / N_ENT) - mean * mean
    scale = gamma_ref[...] * jax.lax.rsqrt(var + 1e-5)
    out_ref[...] = jnp.tanh((x_ref[...] - mean) * scale + beta_ref[...])


def _dense_stage(rel_in_p, rel_out_p, w_in, w_out, gamma, beta,
                 s_parts, deg_in, deg_out):
    nb = N_ENT // ROW_BLK
    sblk = pl.BlockSpec((ROW_BLK, D), lambda i: (i, 0))
    x, stats = pl.pallas_call(
        _mm_bn_kernel,
        grid=(nb,),
        in_specs=[
            pl.BlockSpec((512, D), lambda i: (0, 0)),
            pl.BlockSpec((512, D), lambda i: (0, 0)),
            pl.BlockSpec((D, D), lambda i: (0, 0)),
            pl.BlockSpec((D, D), lambda i: (0, 0)),
        ] + [sblk] * 8 + [
            pl.BlockSpec((ROW_BLK, 1), lambda i: (i, 0)),
            pl.BlockSpec((ROW_BLK, 1), lambda i: (i, 0)),
        ],
        out_specs=[
            pl.BlockSpec((ROW_BLK, D), lambda i: (i, 0)),
            pl.BlockSpec((2, D), lambda i: (0, 0)),
        ],
        out_shape=[
            jax.ShapeDtypeStruct((N_ENT, D), jnp.float32),
            jax.ShapeDtypeStruct((2, D), jnp.float32),
        ],
        scratch_shapes=[
            pltpu.VMEM((512, D), jnp.float32),
            pltpu.VMEM((512, D), jnp.float32),
            pltpu.VMEM((2, D), jnp.float32),
        ],
    )(rel_in_p, rel_out_p, w_in, w_out, *s_parts,
      deg_in[:, None], deg_out[:, None])

    res = pl.pallas_call(
        _bn_apply_kernel,
        grid=(nb,),
        in_specs=[
            pl.BlockSpec((ROW_BLK, D), lambda i: (i, 0)),
            pl.BlockSpec((2, D), lambda i: (0, 0)),
            pl.BlockSpec((D,), lambda i: (0,)),
            pl.BlockSpec((D,), lambda i: (0,)),
        ],
        out_specs=pl.BlockSpec((ROW_BLK, D), lambda i: (i, 0)),
        out_shape=jax.ShapeDtypeStruct((N_ENT, D), jnp.float32),
    )(x, stats, gamma, beta)
    return res


def kernel(rel_embed, rel_embed_in, rel_embed_out, w_in, w_out, gamma, beta,
           edge_index, edge_type):
    outs = _sc_scatter(edge_index, edge_type)
    s_parts = [o.reshape(NPAD, D) for o in outs[:8]]
    deg = outs[8]
    deg_in = deg[:N_ENT]
    deg_out = deg[DEG_PT:DEG_PT + N_ENT]
    pad = ((0, 512 - N_REL), (0, 0))
    rel_in_p = jnp.pad(rel_embed_in, pad)
    rel_out_p = jnp.pad(rel_embed_out, pad)
    res = _dense_stage(rel_in_p, rel_out_p, w_in, w_out, gamma, beta,
                       s_parts, deg_in, deg_out)
    return (res, rel_embed)
